# baseline (speedup 1.0000x reference)
import functools

import numpy as np
import jax
import jax.numpy as jnp
from jax import lax
from jax.experimental import pallas as pl
from jax.experimental.pallas import tpu as pltpu
from jax.experimental.pallas import tpu_sc as plsc

_SEGMENTS = ((0, 128, 1), (128, 128, 3), (512, 128, 5), (1152, 128, 7))
_D = 2048
_Z = 16384
_NW = 32
_ROWS_PER_W = _Z // _NW
_B = 8
_BW = _B * _D
_NBLK = _ROWS_PER_W // _B


def _build_perm() -> np.ndarray:
    p = np.empty(_D, np.int32)
    for off, u, v in _SEGMENTS:
        for k in range(v):
            for i in range(u):
                p[off + k * u + i] = off + i * v + k
    return p


_PERM = _build_perm()


@jax.jit
def _sc_transpose(x, perm):
    mesh = plsc.VectorSubcoreMesh(core_axis_name="c", subcore_axis_name="s")

    @functools.partial(
        pl.kernel,
        mesh=mesh,
        out_type=jax.ShapeDtypeStruct((_Z, _D), jnp.float32),
        scratch_types=[
            pltpu.VMEM((_D,), jnp.int32),
            pltpu.VMEM((_B, _D), jnp.float32),
            pltpu.VMEM((_B, _D), jnp.float32),
            pltpu.VMEM((_B, _D), jnp.float32),
            pltpu.VMEM((_B, _D), jnp.float32),
            pltpu.SemaphoreType.DMA,
            pltpu.SemaphoreType.DMA,
            pltpu.SemaphoreType.DMA,
            pltpu.SemaphoreType.DMA,
        ],
        compiler_params=pltpu.CompilerParams(
            needs_layout_passes=False, use_tc_tiling_on_sc=False),
    )
    def k(x_hbm, perm_hbm, out_hbm, perm_v,
          in_v0, in_v1, out_v0, out_v1,
          sem_in0, sem_in1, sem_out0, sem_out1):
        cid = lax.axis_index("c")
        sid = lax.axis_index("s")
        wid = sid * 2 + cid
        sem_in = (sem_in0, sem_in1)
        sem_out = (sem_out0, sem_out1)
        in_v = (in_v0, in_v1)
        out_v = (out_v0, out_v1)
        row0 = wid * _ROWS_PER_W

        pltpu.sync_copy(perm_hbm, perm_v)

        def in_copy(g, b):
            return pltpu.make_async_copy(
                x_hbm.at[pl.ds(row0 + g * _B, _B)], in_v[b], sem_in[b])

        def out_copy(g, b):
            return pltpu.make_async_copy(
                out_v[b], out_hbm.at[pl.ds(row0 + g * _B, _B)], sem_out[b])

        def compute(b):
            for r in range(_B):
                @plsc.parallel_loop(0, _D, 16, unroll=16)
                def _(i):
                    rv = jnp.full((16,), r, jnp.int32)
                    cols = perm_v[pl.ds(i, 16)]
                    vals = plsc.load_gather(in_v[b], [rv, cols])
                    out_v[b][r, pl.ds(i, 16)] = vals

        in_copy(0, 0).start()
        in_copy(1, 1).start()

        def body(h, carry):
            for b in range(2):
                g = h * 2 + b
                in_copy(g, b).wait()

                @pl.when(g >= 2)
                def _():
                    out_copy(g - 2, b).wait()

                compute(b)
                out_copy(g, b).start()

                @pl.when(g + 2 < _NBLK)
                def _():
                    in_copy(g + 2, b).start()
            return carry

        lax.fori_loop(0, _NBLK // 2, body, 0)
        out_copy(_NBLK - 2, 0).wait()
        out_copy(_NBLK - 1, 1).wait()

    return k(x, perm)


def kernel(x):
    return _sc_transpose(x, jnp.asarray(_PERM))


# speedup vs baseline: 1.5023x; 1.5023x over previous
import functools

import numpy as np
import jax
import jax.numpy as jnp
from jax import lax
from jax.experimental import pallas as pl
from jax.experimental.pallas import tpu as pltpu
from jax.experimental.pallas import tpu_sc as plsc

_SEGMENTS = ((0, 128, 1), (128, 128, 3), (512, 128, 5), (1152, 128, 7))
_D = 2048
_Z = 16384
_NW = 32
_ROWS_PER_W = _Z // _NW
_B = 8
_BW = _B * _D
_NBLK = _ROWS_PER_W // _B


def _build_tab() -> np.ndarray:
    perm = np.empty(_D, np.int32)
    for off, u, v in _SEGMENTS:
        for k in range(v):
            for i in range(u):
                perm[off + k * u + i] = off + i * v + k
    o = np.arange(_BW)
    row, c_out = np.divmod(o, _D)
    c_in = perm[c_out]
    return np.concatenate([row, c_in]).astype(np.int32)


_TAB = _build_tab()


@jax.jit
def _sc_transpose(x, tab):
    mesh = plsc.VectorSubcoreMesh(core_axis_name="c", subcore_axis_name="s")

    @functools.partial(
        pl.kernel,
        mesh=mesh,
        out_type=jax.ShapeDtypeStruct((_Z, _D), jnp.float32),
        scratch_types=[
            pltpu.VMEM((2 * _BW,), jnp.int32),
            pltpu.VMEM((_B, _D), jnp.float32),
            pltpu.VMEM((_B, _D), jnp.float32),
            pltpu.VMEM((_B, _D), jnp.float32),
            pltpu.VMEM((_B, _D), jnp.float32),
            pltpu.SemaphoreType.DMA,
            pltpu.SemaphoreType.DMA,
            pltpu.SemaphoreType.DMA,
            pltpu.SemaphoreType.DMA,
        ],
        compiler_params=pltpu.CompilerParams(needs_layout_passes=False),
    )
    def k(x_hbm, tab_hbm, out_hbm, idx_tab,
          in_v0, in_v1, out_v0, out_v1,
          sem_in0, sem_in1, sem_out0, sem_out1):
        cid = lax.axis_index("c")
        sid = lax.axis_index("s")
        wid = sid * 2 + cid
        sem_in = (sem_in0, sem_in1)
        sem_out = (sem_out0, sem_out1)
        in_v = (in_v0, in_v1)
        out_v = (out_v0, out_v1)
        row0 = wid * _ROWS_PER_W

        pltpu.sync_copy(tab_hbm, idx_tab)

        def in_copy(g, b):
            return pltpu.make_async_copy(
                x_hbm.at[pl.ds(row0 + g * _B, _B)], in_v[b], sem_in[b])

        def out_copy(g, b):
            return pltpu.make_async_copy(
                out_v[b], out_hbm.at[pl.ds(row0 + g * _B, _B)], sem_out[b])

        def compute(b):
            for r in range(_B):
                @plsc.parallel_loop(0, _D, 16, unroll=16)
                def _(i):
                    src0 = idx_tab[pl.ds(r * _D + i, 16)]
                    src1 = idx_tab[pl.ds(_BW + r * _D + i, 16)]
                    vals = plsc.load_gather(in_v[b], [src0, src1])
                    out_v[b][r, pl.ds(i, 16)] = vals

        in_copy(0, 0).start()
        in_copy(1, 1).start()

        def body(h, carry):
            for b in range(2):
                g = h * 2 + b
                in_copy(g, b).wait()

                @pl.when(g >= 2)
                def _():
                    out_copy(g - 2, b).wait()

                compute(b)
                out_copy(g, b).start()

                @pl.when(g + 2 < _NBLK)
                def _():
                    in_copy(g + 2, b).start()
            return carry

        lax.fori_loop(0, _NBLK // 2, body, 0)
        out_copy(_NBLK - 2, 0).wait()
        out_copy(_NBLK - 1, 1).wait()

    return k(x, tab)


def kernel(x):
    return _sc_transpose(x, jnp.asarray(_TAB))


# speedup vs baseline: 2.3662x; 1.5751x over previous
import functools

import numpy as np
import jax
import jax.numpy as jnp
from jax import lax
from jax.experimental import pallas as pl
from jax.experimental.pallas import tpu as pltpu
from jax.experimental.pallas import tpu_sc as plsc

_SEGMENTS = ((0, 128, 1), (128, 128, 3), (512, 128, 5), (1152, 128, 7))
_D = 2048
_Z = 16384
_NW = 32
_ROWS_PER_W = _Z // _NW
_B = 8
_BW = _B * _D
_NBLK = _ROWS_PER_W // _B


def _build_tab() -> np.ndarray:
    perm = np.empty(_D, np.int32)
    for off, u, v in _SEGMENTS:
        for k in range(v):
            for i in range(u):
                perm[off + k * u + i] = off + i * v + k
    return perm


_TAB = _build_tab()


@jax.jit
def _sc_transpose(x, tab):
    mesh = plsc.VectorSubcoreMesh(core_axis_name="c", subcore_axis_name="s")

    @functools.partial(
        pl.kernel,
        mesh=mesh,
        out_type=jax.ShapeDtypeStruct((_Z, _D), jnp.float32),
        scratch_types=[
            pltpu.VMEM((_D,), jnp.int32),
            pltpu.VMEM((_B, _D), jnp.float32),
            pltpu.VMEM((_B, _D), jnp.float32),
            pltpu.VMEM((_B, _D), jnp.float32),
            pltpu.VMEM((_B, _D), jnp.float32),
            pltpu.SemaphoreType.DMA,
            pltpu.SemaphoreType.DMA,
            pltpu.SemaphoreType.DMA,
            pltpu.SemaphoreType.DMA,
        ],
        compiler_params=pltpu.CompilerParams(needs_layout_passes=False),
    )
    def k(x_hbm, tab_hbm, out_hbm, idx_tab,
          in_v0, in_v1, out_v0, out_v1,
          sem_in0, sem_in1, sem_out0, sem_out1):
        cid = lax.axis_index("c")
        sid = lax.axis_index("s")
        wid = sid * 2 + cid
        sem_in = (sem_in0, sem_in1)
        sem_out = (sem_out0, sem_out1)
        in_v = (in_v0, in_v1)
        out_v = (out_v0, out_v1)
        row0 = wid * _ROWS_PER_W

        pltpu.sync_copy(tab_hbm, idx_tab)

        def in_copy(g, b):
            return pltpu.make_async_copy(
                x_hbm.at[pl.ds(row0 + g * _B, _B)], in_v[b], sem_in[b])

        def out_copy(g, b):
            return pltpu.make_async_copy(
                out_v[b], out_hbm.at[pl.ds(row0 + g * _B, _B)], sem_out[b])

        def compute(b):
            for r in range(_B):
                @plsc.parallel_loop(0, _D, 16, unroll=16)
                def _(i):
                    rv = jnp.full((16,), r, jnp.int32)
                    cols = idx_tab[pl.ds(i, 16)]
                    vals = plsc.load_gather(in_v[b], [rv, cols])
                    out_v[b][r, pl.ds(i, 16)] = vals

        in_copy(0, 0).start()
        in_copy(1, 1).start()

        def body(h, carry):
            for b in range(2):
                g = h * 2 + b
                in_copy(g, b).wait()

                @pl.when(g >= 2)
                def _():
                    out_copy(g - 2, b).wait()

                compute(b)
                out_copy(g, b).start()

                @pl.when(g + 2 < _NBLK)
                def _():
                    in_copy(g + 2, b).start()
            return carry

        lax.fori_loop(0, _NBLK // 2, body, 0)
        out_copy(_NBLK - 2, 0).wait()
        out_copy(_NBLK - 1, 1).wait()

    return k(x, tab)


def kernel(x):
    return _sc_transpose(x, jnp.asarray(_TAB))


# speedup vs baseline: 3.0666x; 1.2960x over previous
import functools

import numpy as np
import jax
import jax.numpy as jnp
from jax import lax
from jax.experimental import pallas as pl
from jax.experimental.pallas import tpu as pltpu
from jax.experimental.pallas import tpu_sc as plsc

_SEGMENTS = ((0, 128, 1), (128, 128, 3), (512, 128, 5), (1152, 128, 7))
_D = 2048
_Z = 16384
_NW = 32
_ROWS_PER_W = _Z // _NW
_B = 8
_BW = _B * _D
_NBLK = _ROWS_PER_W // _B


def _build_tab() -> np.ndarray:
    perm = np.empty(_D, np.int32)
    for off, u, v in _SEGMENTS:
        for k in range(v):
            for i in range(u):
                perm[off + k * u + i] = off + i * v + k
    return perm


_TAB = _build_tab()


@jax.jit
def _sc_transpose(x, tab):
    mesh = plsc.VectorSubcoreMesh(core_axis_name="c", subcore_axis_name="s")

    @functools.partial(
        pl.kernel,
        mesh=mesh,
        out_type=jax.ShapeDtypeStruct((_Z, _D), jnp.float32),
        scratch_types=[
            pltpu.VMEM((_D,), jnp.int32),
            pltpu.VMEM((_B, _D), jnp.float32),
            pltpu.VMEM((_B, _D), jnp.float32),
            pltpu.VMEM((_B, _D), jnp.float32),
            pltpu.VMEM((_B, _D), jnp.float32),
            pltpu.SemaphoreType.DMA,
            pltpu.SemaphoreType.DMA,
            pltpu.SemaphoreType.DMA,
            pltpu.SemaphoreType.DMA,
        ],
        compiler_params=pltpu.CompilerParams(needs_layout_passes=False),
    )
    def k(x_hbm, tab_hbm, out_hbm, idx_tab,
          in_v0, in_v1, out_v0, out_v1,
          sem_in0, sem_in1, sem_out0, sem_out1):
        cid = lax.axis_index("c")
        sid = lax.axis_index("s")
        wid = sid * 2 + cid
        sem_in = (sem_in0, sem_in1)
        sem_out = (sem_out0, sem_out1)
        in_v = (in_v0, in_v1)
        out_v = (out_v0, out_v1)
        row0 = wid * _ROWS_PER_W

        pltpu.sync_copy(tab_hbm, idx_tab)

        def in_copy(g, b):
            return pltpu.make_async_copy(
                x_hbm.at[pl.ds(row0 + g * _B, _B)], in_v[b], sem_in[b])

        def out_copy(g, b):
            return pltpu.make_async_copy(
                out_v[b], out_hbm.at[pl.ds(row0 + g * _B, _B)], sem_out[b])

        def compute(b):
            @plsc.parallel_loop(0, _D, 16, unroll=4)
            def _(i):
                cols = idx_tab[pl.ds(i, 16)]
                for r in range(_B):
                    rv = jnp.full((16,), r, jnp.int32)
                    vals = plsc.load_gather(in_v[b], [rv, cols])
                    out_v[b][r, pl.ds(i, 16)] = vals

        in_copy(0, 0).start()
        in_copy(1, 1).start()

        def body(h, carry):
            for b in range(2):
                g = h * 2 + b
                in_copy(g, b).wait()

                @pl.when(g >= 2)
                def _():
                    out_copy(g - 2, b).wait()

                compute(b)
                out_copy(g, b).start()

                @pl.when(g + 2 < _NBLK)
                def _():
                    in_copy(g + 2, b).start()
            return carry

        lax.fori_loop(0, _NBLK // 2, body, 0)
        out_copy(_NBLK - 2, 0).wait()
        out_copy(_NBLK - 1, 1).wait()

    return k(x, tab)


def kernel(x):
    return _sc_transpose(x, jnp.asarray(_TAB))


# speedup vs baseline: 3.1261x; 1.0194x over previous
import functools

import numpy as np
import jax
import jax.numpy as jnp
from jax import lax
from jax.experimental import pallas as pl
from jax.experimental.pallas import tpu as pltpu
from jax.experimental.pallas import tpu_sc as plsc

_SEGMENTS = ((0, 128, 1), (128, 128, 3), (512, 128, 5), (1152, 128, 7))
_D = 2048
_Z = 16384
_NW = 32
_ROWS_PER_W = _Z // _NW
_B = 8
_NBLK = _ROWS_PER_W // _B
_NBUF = 3


def _build_perm() -> np.ndarray:
    p = np.empty(_D, np.int32)
    for off, u, v in _SEGMENTS:
        for k in range(v):
            for i in range(u):
                p[off + k * u + i] = off + i * v + k
    return p


_PERM = _build_perm()


@jax.jit
def _sc_transpose(x, perm):
    mesh = plsc.VectorSubcoreMesh(core_axis_name="c", subcore_axis_name="s")

    @functools.partial(
        pl.kernel,
        mesh=mesh,
        out_type=jax.ShapeDtypeStruct((_Z, _D), jnp.float32),
        scratch_types=(
            [pltpu.VMEM((_D,), jnp.int32)]
            + [pltpu.VMEM((_B, _D), jnp.float32)] * (2 * _NBUF)
            + [pltpu.SemaphoreType.DMA] * (2 * _NBUF)
        ),
        compiler_params=pltpu.CompilerParams(needs_layout_passes=False),
    )
    def k(x_hbm, perm_hbm, out_hbm, idx_tab, *bufs):
        in_v = bufs[:_NBUF]
        out_v = bufs[_NBUF:2 * _NBUF]
        sem_in = bufs[2 * _NBUF:3 * _NBUF]
        sem_out = bufs[3 * _NBUF:4 * _NBUF]
        cid = lax.axis_index("c")
        sid = lax.axis_index("s")
        wid = sid * 2 + cid
        row0 = wid * _ROWS_PER_W

        pltpu.sync_copy(perm_hbm, idx_tab)

        def in_copy(g, b):
            return pltpu.make_async_copy(
                x_hbm.at[pl.ds(row0 + g * _B, _B)], in_v[b], sem_in[b])

        def out_copy(g, b):
            return pltpu.make_async_copy(
                out_v[b], out_hbm.at[pl.ds(row0 + g * _B, _B)], sem_out[b])

        def compute(b):
            @plsc.parallel_loop(0, _D, 16, unroll=4)
            def _(i):
                cols = idx_tab[pl.ds(i, 16)]
                for r in range(_B):
                    rv = jnp.full((16,), r, jnp.int32)
                    vals = plsc.load_gather(in_v[b], [rv, cols])
                    out_v[b][r, pl.ds(i, 16)] = vals

        for b in range(_NBUF):
            in_copy(b, b).start()

        def body(h, carry):
            for b in range(_NBUF):
                g = h * _NBUF + b
                in_copy(g, b).wait()

                @pl.when(g >= _NBUF)
                def _():
                    out_copy(g - _NBUF, b).wait()

                compute(b)
                out_copy(g, b).start()

                @pl.when(g + _NBUF < _NBLK)
                def _():
                    in_copy(g + _NBUF, b).start()
            return carry

        lax.fori_loop(0, _NBLK // _NBUF, body, 0)
        for g in range((_NBLK // _NBUF) * _NBUF, _NBLK):
            b = g % _NBUF
            in_copy(g, b).wait()
            if g >= _NBUF:
                out_copy(g - _NBUF, b).wait()
            compute(b)
            out_copy(g, b).start()
        for g in range(_NBLK - _NBUF, _NBLK):
            out_copy(g, g % _NBUF).wait()

    return k(x, perm)


def kernel(x):
    return _sc_transpose(x, jnp.asarray(_PERM))
